# Initial kernel scaffold; baseline (speedup 1.0000x reference)
import functools

import jax
import jax.numpy as jnp
from jax import lax
from jax.experimental import pallas as pl
from jax.experimental.pallas import tpu as pltpu
from jax.experimental.pallas import tpu_sc as plsc

EMBED_DIM = 64
OUT_DIM = EMBED_DIM + 2

_INFO = plsc.get_sparse_core_info()
NC = _INFO.num_cores
NS = _INFO.num_subcores
L = _INFO.num_lanes
NW = NC * NS

CHUNK = 128


def _body(idx_hbm, emb_hbm, iw_hbm, hash_hbm, out_hbm,
          idx_v, h2_v, w2_v, hflat_v, e_v, out_v, sem, n_tokens):
    wid = lax.axis_index("s") * NC + lax.axis_index("c")
    per_w = n_tokens // NW
    n_chunks = per_w // CHUNK
    lane = lax.iota(jnp.int32, L)

    def chunk_body(j, carry):
        base = wid * per_w + j * CHUNK
        pltpu.sync_copy(idx_hbm.at[pl.ds(base, CHUNK)], idx_v)
        cp_h = pltpu.async_copy(hash_hbm.at[idx_v], h2_v, sem)
        cp_w = pltpu.async_copy(iw_hbm.at[idx_v], w2_v, sem)
        cp_h.wait()
        cp_w.wait()
        for g in range(2 * CHUNK // L):
            p = g * L + lane
            v = plsc.load_gather(h2_v, [p >> 1, p & 1])
            hflat_v[(g * L) // CHUNK, pl.ds((g * L) % CHUNK, L)] = v
        cps = [
            pltpu.async_copy(emb_hbm.at[hflat_v.at[k]],
                             e_v.at[pl.ds(k * CHUNK, CHUNK)], sem)
            for k in range(2)
        ]
        for cp in cps:
            cp.wait()

        def tok_body(t, carry2):
            w0 = plsc.load_gather(w2_v, [jnp.full((L,), t, jnp.int32),
                                         jnp.zeros((L,), jnp.int32)])
            w1 = plsc.load_gather(w2_v, [jnp.full((L,), t, jnp.int32),
                                         jnp.ones((L,), jnp.int32)])
            for q in range(EMBED_DIM // L):
                e0 = e_v[2 * t, pl.ds(q * L, L)]
                e1 = e_v[2 * t + 1, pl.ds(q * L, L)]
                out_v[t, pl.ds(q * L, L)] = w0 * e0 + w1 * e1
            return carry2

        lax.fori_loop(0, CHUNK, tok_body, 0, unroll=4)

        for g in range(CHUNK // L):
            tok = g * L + lane
            w0v = plsc.load_gather(w2_v, [tok, jnp.zeros((L,), jnp.int32)])
            w1v = plsc.load_gather(w2_v, [tok, jnp.ones((L,), jnp.int32)])
            plsc.store_scatter(out_v, [tok, jnp.full((L,), EMBED_DIM, jnp.int32)], w0v)
            plsc.store_scatter(out_v, [tok, jnp.full((L,), EMBED_DIM + 1, jnp.int32)], w1v)
        pltpu.sync_copy(out_v, out_hbm.at[pl.ds(base, CHUNK)])
        return carry

    lax.fori_loop(0, n_chunks, chunk_body, 0)


def kernel(indices, shared_embeddings, importance_weights, hashes):
    batch, seq = indices.shape
    n_tokens = batch * seq
    idx_flat = indices.reshape(n_tokens)

    mesh = plsc.VectorSubcoreMesh(core_axis_name="c", subcore_axis_name="s")
    k = functools.partial(
        pl.kernel,
        out_type=jax.ShapeDtypeStruct((n_tokens, OUT_DIM), jnp.float32),
        mesh=mesh,
        scratch_types=[
            pltpu.VMEM((CHUNK,), jnp.int32),
            pltpu.VMEM((CHUNK, 2), jnp.int32),
            pltpu.VMEM((CHUNK, 2), jnp.float32),
            pltpu.VMEM((2, CHUNK), jnp.int32),
            pltpu.VMEM((2 * CHUNK, EMBED_DIM), jnp.float32),
            pltpu.VMEM((CHUNK, OUT_DIM), jnp.float32),
            pltpu.SemaphoreType.DMA,
        ],
    )(functools.partial(_body, n_tokens=n_tokens))

    out = k(idx_flat, shared_embeddings, importance_weights, hashes)
    return out.reshape(batch, seq, OUT_DIM)


# speedup vs baseline: 2.4824x; 2.4824x over previous
import functools

import jax
import jax.numpy as jnp
from jax import lax
from jax.experimental import pallas as pl
from jax.experimental.pallas import tpu as pltpu
from jax.experimental.pallas import tpu_sc as plsc

EMBED_DIM = 64
OUT_DIM = EMBED_DIM + 2

_INFO = plsc.get_sparse_core_info()
NC = _INFO.num_cores
NS = _INFO.num_subcores
L = _INFO.num_lanes
NW = NC * NS

CHUNK = 128


def _body(idx_hbm, emb_hbm, iwf_hbm, hashf_hbm, out_hbm,
          idx_v, ix0_v, ix1_v, h0_v, h1_v, w0_v, w1_v,
          e0_v, e1_v, out_v, sem, n_tokens):
    wid = lax.axis_index("s") * NC + lax.axis_index("c")
    per_w = n_tokens // NW
    n_chunks = per_w // CHUNK
    lane = lax.iota(jnp.int32, L)

    def chunk_body(j, carry):
        base = wid * per_w + j * CHUNK
        pltpu.sync_copy(idx_hbm.at[pl.ds(base, CHUNK)], idx_v)
        for g in range(CHUNK // L):
            v = idx_v[pl.ds(g * L, L)]
            ix0_v[pl.ds(g * L, L)] = v * 2
            ix1_v[pl.ds(g * L, L)] = v * 2 + 1
        cps = [
            pltpu.async_copy(hashf_hbm.at[ix0_v], h0_v, sem),
            pltpu.async_copy(hashf_hbm.at[ix1_v], h1_v, sem),
            pltpu.async_copy(iwf_hbm.at[ix0_v], w0_v, sem),
            pltpu.async_copy(iwf_hbm.at[ix1_v], w1_v, sem),
        ]
        for cp in cps:
            cp.wait()
        cps = [
            pltpu.async_copy(emb_hbm.at[h0_v], e0_v, sem),
            pltpu.async_copy(emb_hbm.at[h1_v], e1_v, sem),
        ]
        for cp in cps:
            cp.wait()

        def group_body(g, carry2):
            w0g = w0_v[pl.ds(g * L, L)]
            w1g = w1_v[pl.ds(g * L, L)]
            for tl in range(L):
                t = g * L + tl
                w0 = jnp.full((L,), w0g[tl])
                w1 = jnp.full((L,), w1g[tl])
                for q in range(EMBED_DIM // L):
                    e0 = e0_v[t, pl.ds(q * L, L)]
                    e1 = e1_v[t, pl.ds(q * L, L)]
                    out_v[pl.ds(t * OUT_DIM + q * L, L)] = w0 * e0 + w1 * e1
                wvec = jnp.where(lane == 0, w0, jnp.where(lane == 1, w1, 0.0))
                out_v[pl.ds(t * OUT_DIM + EMBED_DIM, L)] = wvec
            return carry2

        lax.fori_loop(0, CHUNK // L, group_body, 0)

        pltpu.sync_copy(out_v.at[pl.ds(0, CHUNK * OUT_DIM)],
                        out_hbm.at[pl.ds(base * OUT_DIM, CHUNK * OUT_DIM)])
        return carry

    lax.fori_loop(0, n_chunks, chunk_body, 0)


def kernel(indices, shared_embeddings, importance_weights, hashes):
    batch, seq = indices.shape
    n_tokens = batch * seq
    idx_flat = indices.reshape(n_tokens)
    hashes_flat = hashes.reshape(-1)
    iw_flat = importance_weights.reshape(-1)

    mesh = plsc.VectorSubcoreMesh(core_axis_name="c", subcore_axis_name="s")
    k = functools.partial(
        pl.kernel,
        out_type=jax.ShapeDtypeStruct((n_tokens * OUT_DIM,), jnp.float32),
        mesh=mesh,
        compiler_params=pltpu.CompilerParams(use_tc_tiling_on_sc=False),
        scratch_types=[
            pltpu.VMEM((CHUNK,), jnp.int32),
            pltpu.VMEM((CHUNK,), jnp.int32),
            pltpu.VMEM((CHUNK,), jnp.int32),
            pltpu.VMEM((CHUNK,), jnp.int32),
            pltpu.VMEM((CHUNK,), jnp.int32),
            pltpu.VMEM((CHUNK,), jnp.float32),
            pltpu.VMEM((CHUNK,), jnp.float32),
            pltpu.VMEM((CHUNK, EMBED_DIM), jnp.float32),
            pltpu.VMEM((CHUNK, EMBED_DIM), jnp.float32),
            pltpu.VMEM((CHUNK * OUT_DIM + L,), jnp.float32),
            pltpu.SemaphoreType.DMA,
        ],
    )(functools.partial(_body, n_tokens=n_tokens))

    out = k(idx_flat, shared_embeddings, iw_flat, hashes_flat)
    return out.reshape(batch, seq, OUT_DIM)


# speedup vs baseline: 5.6865x; 2.2907x over previous
import functools

import jax
import jax.numpy as jnp
from jax import lax
from jax.experimental import pallas as pl
from jax.experimental.pallas import tpu as pltpu
from jax.experimental.pallas import tpu_sc as plsc

EMBED_DIM = 64
OUT_DIM = EMBED_DIM + 2

_INFO = plsc.get_sparse_core_info()
NC = _INFO.num_cores
NS = _INFO.num_subcores
L = _INFO.num_lanes
NW = NC * NS

CHUNK = 128
OUT_WORDS = CHUNK * OUT_DIM


def _body(idx_hbm, emb_hbm, h0_hbm, h1_hbm, w0_hbm, w1_hbm, out_hbm,
          idx_v, h0_v, h1_v, w0_v, w1_v, e0_v, e1_v, out_v,
          idx_sem, hw_sem, e_sem, out_sem, n_tokens):
    wid = lax.axis_index("s") * NC + lax.axis_index("c")
    per_w = n_tokens // NW
    n_chunks = per_w // CHUNK
    lane = lax.iota(jnp.int32, L)

    def base_of(j):
        return wid * per_w + j * CHUNK

    def start_idx(j, p):
        pltpu.async_copy(idx_hbm.at[pl.ds(base_of(j), CHUNK)],
                         idx_v.at[p], idx_sem.at[p])

    def wait_idx(p):
        pltpu.make_async_copy(idx_hbm.at[pl.ds(0, CHUNK)],
                              idx_v.at[p], idx_sem.at[p]).wait()

    def start_hw(j, p):
        del j
        pltpu.async_copy(h0_hbm.at[idx_v.at[p]], h0_v.at[p], hw_sem.at[p])
        pltpu.async_copy(h1_hbm.at[idx_v.at[p]], h1_v.at[p], hw_sem.at[p])
        pltpu.async_copy(w0_hbm.at[idx_v.at[p]], w0_v.at[p], hw_sem.at[p])
        pltpu.async_copy(w1_hbm.at[idx_v.at[p]], w1_v.at[p], hw_sem.at[p])

    def wait_hw(p):
        pltpu.make_async_copy(h0_hbm.at[idx_v.at[p]], h0_v.at[p], hw_sem.at[p]).wait()
        pltpu.make_async_copy(h1_hbm.at[idx_v.at[p]], h1_v.at[p], hw_sem.at[p]).wait()
        pltpu.make_async_copy(w0_hbm.at[idx_v.at[p]], w0_v.at[p], hw_sem.at[p]).wait()
        pltpu.make_async_copy(w1_hbm.at[idx_v.at[p]], w1_v.at[p], hw_sem.at[p]).wait()

    def start_e(j, p):
        del j
        pltpu.async_copy(emb_hbm.at[h0_v.at[p]], e0_v.at[p], e_sem.at[p])
        pltpu.async_copy(emb_hbm.at[h1_v.at[p]], e1_v.at[p], e_sem.at[p])

    def wait_e(p):
        pltpu.make_async_copy(emb_hbm.at[h0_v.at[p]], e0_v.at[p], e_sem.at[p]).wait()
        pltpu.make_async_copy(emb_hbm.at[h1_v.at[p]], e1_v.at[p], e_sem.at[p]).wait()

    def start_out(j, p):
        pltpu.async_copy(out_v.at[p, pl.ds(0, OUT_WORDS)],
                         out_hbm.at[pl.ds(base_of(j) * OUT_DIM, OUT_WORDS)],
                         out_sem.at[p])

    def wait_out(p):
        pltpu.make_async_copy(out_v.at[p, pl.ds(0, OUT_WORDS)],
                              out_hbm.at[pl.ds(0, OUT_WORDS)],
                              out_sem.at[p]).wait()

    def compute(p):
        def group_body(g, carry):
            w0g = w0_v[p, pl.ds(g * L, L)]
            w1g = w1_v[p, pl.ds(g * L, L)]
            for tl in range(L):
                t = g * L + tl
                w0 = jnp.full((L,), w0g[tl])
                w1 = jnp.full((L,), w1g[tl])
                for q in range(EMBED_DIM // L):
                    e0 = e0_v[p, t, pl.ds(q * L, L)]
                    e1 = e1_v[p, t, pl.ds(q * L, L)]
                    out_v[p, pl.ds(t * OUT_DIM + q * L, L)] = w0 * e0 + w1 * e1
                wvec = jnp.where(lane == 0, w0, jnp.where(lane == 1, w1, 0.0))
                out_v[p, pl.ds(t * OUT_DIM + EMBED_DIM, L)] = wvec
            return carry

        lax.fori_loop(0, CHUNK // L, group_body, 0)

    start_idx(0, 0)
    wait_idx(0)
    start_hw(0, 0)
    start_idx(1, 1)
    wait_hw(0)
    start_e(0, 0)
    wait_idx(1)
    start_hw(1, 1)

    start_idx(2, 0)
    wait_e(0)
    compute(0)
    start_out(0, 0)
    wait_hw(1)
    start_e(1, 1)
    wait_idx(0)
    start_hw(2, 0)
    start_idx(3, 1)
    wait_e(1)
    compute(1)
    start_out(1, 1)
    wait_hw(0)
    start_e(2, 0)
    wait_idx(1)
    start_hw(3, 1)

    def main_body(j, carry):
        p = lax.rem(j, 2)
        pn = 1 - p
        start_idx(j + 2, p)
        wait_e(p)
        wait_out(p)
        compute(p)
        start_out(j, p)
        wait_hw(pn)
        start_e(j + 1, pn)
        wait_idx(p)
        start_hw(j + 2, p)
        return carry

    lax.fori_loop(2, n_chunks - 2, main_body, 0)

    jm2 = n_chunks - 2
    p = jm2 % 2
    wait_e(p)
    wait_out(p)
    compute(p)
    start_out(jm2, p)
    wait_hw(1 - p)
    start_e(jm2 + 1, 1 - p)

    p = (n_chunks - 1) % 2
    wait_e(p)
    wait_out(p)
    compute(p)
    start_out(n_chunks - 1, p)

    wait_out(0)
    wait_out(1)


def kernel(indices, shared_embeddings, importance_weights, hashes):
    batch, seq = indices.shape
    n_tokens = batch * seq
    idx_flat = indices.reshape(n_tokens)
    h0 = hashes[:, 0]
    h1 = hashes[:, 1]
    w0 = importance_weights[:, 0]
    w1 = importance_weights[:, 1]

    mesh = plsc.VectorSubcoreMesh(core_axis_name="c", subcore_axis_name="s")
    k = functools.partial(
        pl.kernel,
        out_type=jax.ShapeDtypeStruct((n_tokens * OUT_DIM,), jnp.float32),
        mesh=mesh,
        compiler_params=pltpu.CompilerParams(use_tc_tiling_on_sc=False),
        scratch_types=[
            pltpu.VMEM((2, CHUNK), jnp.int32),
            pltpu.VMEM((2, CHUNK), jnp.int32),
            pltpu.VMEM((2, CHUNK), jnp.int32),
            pltpu.VMEM((2, CHUNK), jnp.float32),
            pltpu.VMEM((2, CHUNK), jnp.float32),
            pltpu.VMEM((2, CHUNK, EMBED_DIM), jnp.float32),
            pltpu.VMEM((2, CHUNK, EMBED_DIM), jnp.float32),
            pltpu.VMEM((2, OUT_WORDS + L), jnp.float32),
            pltpu.SemaphoreType.DMA((2,)),
            pltpu.SemaphoreType.DMA((2,)),
            pltpu.SemaphoreType.DMA((2,)),
            pltpu.SemaphoreType.DMA((2,)),
        ],
    )(functools.partial(_body, n_tokens=n_tokens))

    out = k(idx_flat, shared_embeddings, h0, h1, w0, w1)
    return out.reshape(batch, seq, OUT_DIM)


# speedup vs baseline: 10.5889x; 1.8621x over previous
import functools

import jax
import jax.numpy as jnp
from jax import lax
from jax.experimental import pallas as pl
from jax.experimental.pallas import tpu as pltpu
from jax.experimental.pallas import tpu_sc as plsc

EMBED_DIM = 64
OUT_DIM = EMBED_DIM + 2
ROW_PAD = 128
ROW_LIVE = 80

_INFO = plsc.get_sparse_core_info()
NC = _INFO.num_cores
NS = _INFO.num_subcores
L = _INFO.num_lanes
NW = NC * NS

CHUNK = 128

_GDN = lax.GatherDimensionNumbers(
    offset_dims=(), collapsed_slice_dims=(0,), start_index_map=(0,))


def _dyngather(v, idx):
    return lax.gather(v, idx[:, None], _GDN, (1,),
                      mode=lax.GatherScatterMode.PROMISE_IN_BOUNDS)


def _body(idx_hbm, emb_hbm, h0_hbm, h1_hbm, w0_hbm, w1_hbm, out_hbm,
          idx_v, h0_v, h1_v, w0_v, w1_v, e0_v, e1_v, out_v,
          idx_sem, hw_sem, e_sem, out_sem, n_tokens):
    wid = lax.axis_index("s") * NC + lax.axis_index("c")
    per_w = n_tokens // NW
    n_chunks = per_w // CHUNK
    lane = lax.iota(jnp.int32, L)

    def base_of(j):
        return wid * per_w + j * CHUNK

    def start_idx(j, p):
        pltpu.async_copy(idx_hbm.at[pl.ds(base_of(j), CHUNK)],
                         idx_v.at[p], idx_sem.at[p])

    def wait_idx(p):
        pltpu.make_async_copy(idx_hbm.at[pl.ds(0, CHUNK)],
                              idx_v.at[p], idx_sem.at[p]).wait()

    def start_hw(pidx, phw):
        pltpu.async_copy(h0_hbm.at[idx_v.at[pidx]], h0_v.at[phw], hw_sem.at[phw])
        pltpu.async_copy(h1_hbm.at[idx_v.at[pidx]], h1_v.at[phw], hw_sem.at[phw])
        pltpu.async_copy(w0_hbm.at[idx_v.at[pidx]], w0_v.at[phw], hw_sem.at[phw])
        pltpu.async_copy(w1_hbm.at[idx_v.at[pidx]], w1_v.at[phw], hw_sem.at[phw])

    def wait_hw(phw):
        pltpu.make_async_copy(h0_hbm.at[idx_v.at[0]], h0_v.at[phw], hw_sem.at[phw]).wait()
        pltpu.make_async_copy(h1_hbm.at[idx_v.at[0]], h1_v.at[phw], hw_sem.at[phw]).wait()
        pltpu.make_async_copy(w0_hbm.at[idx_v.at[0]], w0_v.at[phw], hw_sem.at[phw]).wait()
        pltpu.make_async_copy(w1_hbm.at[idx_v.at[0]], w1_v.at[phw], hw_sem.at[phw]).wait()

    def start_e(phw, p2):
        pltpu.async_copy(emb_hbm.at[h0_v.at[phw]], e0_v.at[p2], e_sem.at[p2])
        pltpu.async_copy(emb_hbm.at[h1_v.at[phw]], e1_v.at[p2], e_sem.at[p2])

    def wait_e(p2):
        pltpu.make_async_copy(emb_hbm.at[h0_v.at[0]], e0_v.at[p2], e_sem.at[p2]).wait()
        pltpu.make_async_copy(emb_hbm.at[h1_v.at[0]], e1_v.at[p2], e_sem.at[p2]).wait()

    def start_out(j, p):
        pltpu.async_copy(out_v.at[p],
                         out_hbm.at[pl.ds(base_of(j), CHUNK), pl.ds(0, ROW_LIVE)],
                         out_sem.at[p])

    def wait_out(p):
        pltpu.make_async_copy(out_v.at[p],
                              out_hbm.at[pl.ds(0, CHUNK), pl.ds(0, ROW_LIVE)],
                              out_sem.at[p]).wait()

    def compute(p2, p3):
        def group_body(g, carry):
            w0g = w0_v[p3, pl.ds(g * L, L)]
            w1g = w1_v[p3, pl.ds(g * L, L)]
            for tl in range(L):
                t = g * L + tl
                tsplat = jnp.full((L,), tl, jnp.int32)
                w0 = _dyngather(w0g, tsplat)
                w1 = _dyngather(w1g, tsplat)
                for q in range(EMBED_DIM // L):
                    e0 = e0_v[p2, t, pl.ds(q * L, L)]
                    e1 = e1_v[p2, t, pl.ds(q * L, L)]
                    out_v[p2, t, pl.ds(q * L, L)] = w0 * e0 + w1 * e1
                wvec = jnp.where(lane == 0, w0, jnp.where(lane == 1, w1, 0.0))
                out_v[p2, t, pl.ds(EMBED_DIM, L)] = wvec
            return carry

        lax.fori_loop(0, CHUNK // L, group_body, 0)

    start_idx(0, 0)
    wait_idx(0)
    start_hw(0, 0)
    start_idx(1, 1)
    wait_idx(1)
    start_hw(1, 1)
    start_idx(2, 0)
    wait_hw(0)
    start_e(0, 0)

    for j0 in (0, 1):
        wait_hw((j0 + 1) % 3)
        start_e((j0 + 1) % 3, (j0 + 1) % 2)
        start_idx(j0 + 3, (j0 + 3) % 2)
        wait_idx((j0 + 2) % 2)
        start_hw((j0 + 2) % 2, (j0 + 2) % 3)
        wait_e(j0 % 2)
        compute(j0 % 2, j0 % 3)
        start_out(j0, j0 % 2)

    def main_body(j, carry):
        p2 = lax.rem(j, 2)
        pn2 = 1 - p2
        wait_hw(lax.rem(j + 1, 3))
        start_e(lax.rem(j + 1, 3), pn2)
        start_idx(j + 3, lax.rem(j + 3, 2))
        wait_idx(lax.rem(j + 2, 2))
        start_hw(lax.rem(j + 2, 2), lax.rem(j + 2, 3))
        wait_e(p2)
        wait_out(p2)
        compute(p2, lax.rem(j, 3))
        start_out(j, p2)
        return carry

    lax.fori_loop(2, n_chunks - 3, main_body, 0)

    j = n_chunks - 3
    p2 = j % 2
    wait_hw((j + 1) % 3)
    start_e((j + 1) % 3, 1 - p2)
    wait_idx((j + 2) % 2)
    start_hw((j + 2) % 2, (j + 2) % 3)
    wait_e(p2)
    wait_out(p2)
    compute(p2, j % 3)
    start_out(j, p2)

    j = n_chunks - 2
    p2 = j % 2
    wait_hw((j + 1) % 3)
    start_e((j + 1) % 3, 1 - p2)
    wait_e(p2)
    wait_out(p2)
    compute(p2, j % 3)
    start_out(j, p2)

    j = n_chunks - 1
    p2 = j % 2
    wait_e(p2)
    wait_out(p2)
    compute(p2, j % 3)
    start_out(j, p2)

    wait_out(0)
    wait_out(1)


def kernel(indices, shared_embeddings, importance_weights, hashes):
    batch, seq = indices.shape
    n_tokens = batch * seq
    idx_flat = indices.reshape(n_tokens)
    h0 = hashes[:, 0]
    h1 = hashes[:, 1]
    w0 = importance_weights[:, 0]
    w1 = importance_weights[:, 1]

    mesh = plsc.VectorSubcoreMesh(core_axis_name="c", subcore_axis_name="s")
    k = functools.partial(
        pl.kernel,
        out_type=jax.ShapeDtypeStruct((n_tokens, ROW_PAD), jnp.float32),
        mesh=mesh,
        compiler_params=pltpu.CompilerParams(use_tc_tiling_on_sc=False),
        scratch_types=[
            pltpu.VMEM((2, CHUNK), jnp.int32),
            pltpu.VMEM((3, CHUNK), jnp.int32),
            pltpu.VMEM((3, CHUNK), jnp.int32),
            pltpu.VMEM((3, CHUNK), jnp.float32),
            pltpu.VMEM((3, CHUNK), jnp.float32),
            pltpu.VMEM((2, CHUNK, EMBED_DIM), jnp.float32),
            pltpu.VMEM((2, CHUNK, EMBED_DIM), jnp.float32),
            pltpu.VMEM((2, CHUNK, ROW_LIVE), jnp.float32),
            pltpu.SemaphoreType.DMA((2,)),
            pltpu.SemaphoreType.DMA((3,)),
            pltpu.SemaphoreType.DMA((2,)),
            pltpu.SemaphoreType.DMA((2,)),
        ],
    )(functools.partial(_body, n_tokens=n_tokens))

    out = k(idx_flat, shared_embeddings, h0, h1, w0, w1)
    return out.reshape(batch, seq, ROW_PAD)[..., :OUT_DIM]
